# baseline (speedup 1.0000x reference)
import jax
import jax.numpy as jnp
from jax.experimental import pallas as pl

_TOPK = 8
_BLOCK_T = 1024
_TSPLIT = 8


def _router_block(*refs):
    h_refs = refs[:_TSPLIT]
    w_ref, rw_ref, idx_ref = refs[_TSPLIT:]
    parts = []
    for hr in h_refs:
        parts.append(jax.lax.dot_general(
            w_ref[...], hr[...],
            dimension_numbers=(((1,), (1,)), ((), ())),
            preferred_element_type=jnp.float32,
        ))
    logits = jnp.concatenate(parts, axis=1)
    e, bt = logits.shape
    iota = jax.lax.broadcasted_iota(jnp.int32, (e, bt), 0)
    cur = logits
    vals, idxs = [], []
    for _ in range(_TOPK):
        m = jnp.max(cur, axis=0, keepdims=True)
        idx = jnp.min(jnp.where(cur == m, iota, e), axis=0, keepdims=True)
        vals.append(m)
        idxs.append(idx)
        cur = jnp.where(iota == idx, -jnp.inf, cur)
    v = jnp.concatenate(vals, axis=0)
    ii = jnp.concatenate(idxs, axis=0)
    ex = jnp.exp(v - vals[0])
    rw = ex / jnp.sum(ex, axis=0, keepdims=True)
    rw_ref[...] = rw.T
    idx_ref[...] = ii.T


def _chunk_spec(j, tc, k):
    return pl.BlockSpec((tc, k), lambda i, j=j: (i * _TSPLIT + j, 0))


def kernel(hidden_states, W):
    n, k = hidden_states.shape
    e = W.shape[0]
    tc = _BLOCK_T // _TSPLIT
    in_specs = [_chunk_spec(j, tc, k) for j in range(_TSPLIT)]
    in_specs.append(pl.BlockSpec((e, k), lambda i: (0, 0)))
    rw, idx = pl.pallas_call(
        _router_block,
        grid=(n // _BLOCK_T,),
        in_specs=in_specs,
        out_specs=[
            pl.BlockSpec((_BLOCK_T, _TOPK), lambda i: (i, 0)),
            pl.BlockSpec((_BLOCK_T, _TOPK), lambda i: (i, 0)),
        ],
        out_shape=[
            jax.ShapeDtypeStruct((n, _TOPK), jnp.float32),
            jax.ShapeDtypeStruct((n, _TOPK), jnp.int32),
        ],
    )(*([hidden_states] * _TSPLIT), W)
    return rw, idx
